# baseline (speedup 1.0000x reference)
import functools

import jax
import jax.numpy as jnp
from jax import lax
from jax.experimental import pallas as pl
from jax.experimental.pallas import tpu as pltpu
from jax.experimental.pallas import tpu_sc as plsc

BATCH = 16384
EMBED_DIM = 32
WIDE = 128
PACK = WIDE // EMBED_DIM
NUM_USERS = 1000000
NUM_FILMS = 100000
NUM_CORES = 2
NUM_SUBCORES = 16
NUM_WORKERS = NUM_CORES * NUM_SUBCORES
ROWS_PER_WORKER = BATCH // NUM_WORKERS
CHUNK = 128
NUM_CHUNKS = ROWS_PER_WORKER // CHUNK
LANES = 16
BLK = 512

U_FULL = (NUM_USERS // BLK)
I_FULL = (NUM_FILMS // BLK)


def _relayout_sc(ut_t, it_t):
    mesh = plsc.VectorSubcoreMesh(core_axis_name="c", subcore_axis_name="s")

    @functools.partial(
        pl.kernel,
        out_type=[
            jax.ShapeDtypeStruct((U_FULL * BLK // PACK, WIDE), jnp.float32),
            jax.ShapeDtypeStruct((I_FULL * BLK // PACK, WIDE), jnp.float32),
        ],
        mesh=mesh,
        compiler_params=pltpu.CompilerParams(needs_layout_passes=False),
        scratch_types=[
            pltpu.VMEM((2, EMBED_DIM, BLK), jnp.float32),
            pltpu.VMEM((2, BLK // PACK, WIDE), jnp.float32),
            pltpu.SemaphoreType.DMA,
            pltpu.SemaphoreType.DMA,
        ],
    )
    def relayout_kernel(ut_hbm, it_hbm, uw_hbm, iw_hbm, in_v, out_v,
                        isem, osem):
        wid = lax.axis_index("s") * NUM_CORES + lax.axis_index("c")

        def transpose_block(slot, width):
            def body(q, _):
                for v in range(WIDE // LANES):
                    c16 = lax.iota(jnp.int32, LANES) + (v % 2) * LANES
                    col16 = jnp.full((LANES,), 0, jnp.int32) + (PACK * q + v // 2)
                    vals = plsc.load_gather(in_v.at[slot], [c16, col16])
                    out_v[slot, q, pl.ds(v * LANES, LANES)] = vals
                return ()
            lax.fori_loop(0, width // PACK, body, (), unroll=False)

        def run(tab_hbm, wide_hbm, nfull, per_lo, nlo):
            nblk = per_lo + jnp.where(wid < nlo, 1, 0)
            blk0 = wid * per_lo + jnp.minimum(wid, nlo)

            def in_copy(b, slot):
                col0 = pl.multiple_of((blk0 + b) * BLK, BLK)
                return pltpu.make_async_copy(
                    tab_hbm.at[:, pl.ds(col0, BLK)], in_v.at[slot], isem)

            def out_copy(b, slot):
                row0 = pl.multiple_of((blk0 + b) * (BLK // PACK), BLK // PACK)
                return pltpu.make_async_copy(
                    out_v.at[slot], wide_hbm.at[pl.ds(row0, BLK // PACK)], osem)

            in_copy(0, 0).start()

            def body(b, _):
                slot = lax.rem(b, 2)
                nxt = lax.rem(b + 1, 2)
                in_copy(b, slot).wait()
                in_copy(jnp.minimum(b + 1, nblk - 1), nxt).start()
                @pl.when(b >= 2)
                def _():
                    out_copy(b - 2, slot).wait()
                transpose_block(slot, BLK)
                out_copy(b, slot).start()
                return ()

            lax.fori_loop(0, nblk, body, (), unroll=False)
            in_copy(nblk - 1, lax.rem(nblk, 2)).wait()

            @pl.when(nblk >= 2)
            def _():
                out_copy(nblk - 2, lax.rem(nblk, 2)).wait()
            out_copy(nblk - 1, lax.rem(nblk - 1, 2)).wait()

        run(ut_hbm, uw_hbm, U_FULL, U_FULL // NUM_WORKERS,
            U_FULL % NUM_WORKERS)
        run(it_hbm, iw_hbm, I_FULL, I_FULL // NUM_WORKERS,
            I_FULL % NUM_WORKERS)

    return relayout_kernel(ut_t, it_t)


def _gather_sc(uidx, iidx, ut_wide, it_wide):
    mesh = plsc.VectorSubcoreMesh(core_axis_name="c", subcore_axis_name="s")

    @functools.partial(
        pl.kernel,
        out_type=[
            jax.ShapeDtypeStruct((BATCH, WIDE), jnp.float32),
            jax.ShapeDtypeStruct((BATCH, WIDE), jnp.float32),
        ],
        mesh=mesh,
        scratch_types=[
            pltpu.VMEM((NUM_CHUNKS, CHUNK), jnp.int32),
            pltpu.VMEM((NUM_CHUNKS, CHUNK), jnp.int32),
            pltpu.VMEM((2, CHUNK, WIDE), jnp.float32),
            pltpu.VMEM((2, CHUNK, WIDE), jnp.float32),
            pltpu.SemaphoreType.DMA,
            pltpu.SemaphoreType.DMA,
        ],
    )
    def gather_kernel(uidx_hbm, iidx_hbm, ut_hbm, it_hbm, uw_hbm, iw_hbm,
                      uidx_v, iidx_v, ur_v, ir_v, gsem, osem):
        wid = lax.axis_index("s") * NUM_CORES + lax.axis_index("c")
        base = wid * ROWS_PER_WORKER
        pltpu.sync_copy(uidx_hbm.at[wid], uidx_v)
        pltpu.sync_copy(iidx_hbm.at[wid], iidx_v)

        gathers = [None, None]
        drains = [None, None]

        def fire(j, slot):
            gathers[slot] = (
                pltpu.async_copy(ut_hbm.at[uidx_v.at[j]], ur_v.at[slot], gsem),
                pltpu.async_copy(it_hbm.at[iidx_v.at[j]], ir_v.at[slot], gsem),
            )

        def drain(j, slot):
            for g in gathers[slot]:
                g.wait()
            row0 = base + j * CHUNK
            drains[slot] = (
                pltpu.async_copy(ur_v.at[slot], uw_hbm.at[pl.ds(row0, CHUNK)],
                                 osem),
                pltpu.async_copy(ir_v.at[slot], iw_hbm.at[pl.ds(row0, CHUNK)],
                                 osem),
            )

        for j in range(NUM_CHUNKS):
            slot = j % 2
            if drains[slot] is not None:
                for d in drains[slot]:
                    d.wait()
                drains[slot] = None
            fire(j, slot)
            if j >= 1:
                drain(j - 1, (j - 1) % 2)
        drain(NUM_CHUNKS - 1, (NUM_CHUNKS - 1) % 2)
        for slot in (0, 1):
            if drains[slot] is not None:
                for d in drains[slot]:
                    d.wait()

    return gather_kernel(uidx, iidx, ut_wide, it_wide)


CUT_U = U_FULL * BLK
CUT_I = I_FULL * BLK


def _mlp_body(uw_ref, iw_ref, ur_ref, ir_ref, utail_ref, itail_ref,
              w1_ref, b1_ref, w2_ref, b2_ref,
              w3_ref, b3_ref, w4_ref, b4_ref, o_ref):
    uw = uw_ref[...]
    iw = iw_ref[...]
    ur = ur_ref[...]
    ir = ir_ref[...]
    uo = lax.rem(ur, PACK)
    io = lax.rem(ir, PACK)
    ue = jnp.zeros(uw.shape[:1] + (EMBED_DIM,), jnp.float32)
    ie = ue
    for g in range(PACK):
        sl = slice(g * EMBED_DIM, (g + 1) * EMBED_DIM)
        ue = ue + uw[:, sl] * (uo == g).astype(jnp.float32)
        ie = ie + iw[:, sl] * (io == g).astype(jnp.float32)
    oh_u = (ur - CUT_U == lax.broadcasted_iota(jnp.int32, (1, NUM_USERS - CUT_U), 1)
            ).astype(jnp.float32)
    oh_i = (ir - CUT_I == lax.broadcasted_iota(jnp.int32, (1, NUM_FILMS - CUT_I), 1)
            ).astype(jnp.float32)
    ue_tail = jnp.dot(oh_u, utail_ref[...], preferred_element_type=jnp.float32)
    ie_tail = jnp.dot(oh_i, itail_ref[...], preferred_element_type=jnp.float32)
    ue = jnp.where(ur < CUT_U, ue, ue_tail)
    ie = jnp.where(ir < CUT_I, ie, ie_tail)
    h = (jnp.dot(ue, w1_ref[0:EMBED_DIM, :], preferred_element_type=jnp.float32)
         + jnp.dot(ie, w1_ref[EMBED_DIM:2 * EMBED_DIM, :],
                   preferred_element_type=jnp.float32)
         + b1_ref[...])
    h = jnp.maximum(h, 0.0)
    h = jnp.maximum(jnp.dot(h, w2_ref[...], preferred_element_type=jnp.float32)
                    + b2_ref[...], 0.0)
    h = jnp.maximum(jnp.dot(h, w3_ref[...], preferred_element_type=jnp.float32)
                    + b3_ref[...], 0.0)
    y = jax.nn.sigmoid(jnp.dot(h, w4_ref[...], preferred_element_type=jnp.float32)
                       + b4_ref[...])
    o_ref[...] = y * 5.0 + 1.0


def _mlp_tc(uw, iw, ur, ir, utail, itail, W1, b1, W2, b2, W3, b3, W4, b4):
    blk = 2048
    grid = (BATCH // blk,)
    full = lambda shape: pl.BlockSpec(shape, lambda i: (0,) * len(shape))
    return pl.pallas_call(
        _mlp_body,
        grid=grid,
        in_specs=[
            pl.BlockSpec((blk, WIDE), lambda i: (i, 0)),
            pl.BlockSpec((blk, WIDE), lambda i: (i, 0)),
            pl.BlockSpec((blk, 1), lambda i: (i, 0)),
            pl.BlockSpec((blk, 1), lambda i: (i, 0)),
            full(utail.shape), full(itail.shape),
            full(W1.shape), full(b1.shape),
            full(W2.shape), full(b2.shape),
            full(W3.shape), full(b3.shape),
            full(W4.shape), full(b4.shape),
        ],
        out_specs=pl.BlockSpec((blk, 1), lambda i: (i, 0)),
        out_shape=jax.ShapeDtypeStruct((BATCH, 1), jnp.float32),
    )(uw, iw, ur, ir, utail, itail, W1, b1, W2, b2, W3, b3, W4, b4)


def kernel(user_indices, item_indices, emb_user, emb_item,
           W1, b1, W2, b2, W3, b3, W4, b4):
    ui = user_indices.astype(jnp.int32)
    ii = item_indices.astype(jnp.int32)
    uidx = jnp.minimum(ui // PACK, CUT_U // PACK - 1)
    iidx = jnp.minimum(ii // PACK, CUT_I // PACK - 1)
    ut_wide, it_wide = _relayout_sc(emb_user.T, emb_item.T)
    uw, iw = _gather_sc(uidx.reshape(NUM_WORKERS, NUM_CHUNKS, CHUNK),
                        iidx.reshape(NUM_WORKERS, NUM_CHUNKS, CHUNK),
                        ut_wide, it_wide)
    return _mlp_tc(uw, iw, ui.reshape(BATCH, 1), ii.reshape(BATCH, 1),
                   emb_user[CUT_U:], emb_item[CUT_I:],
                   W1, b1.reshape(1, -1), W2, b2.reshape(1, -1),
                   W3, b3.reshape(1, -1), W4, b4.reshape(1, -1))


# speedup vs baseline: 1.6511x; 1.6511x over previous
import functools

import jax
import jax.numpy as jnp
from jax import lax
from jax.experimental import pallas as pl
from jax.experimental.pallas import tpu as pltpu
from jax.experimental.pallas import tpu_sc as plsc

BATCH = 16384
EMBED_DIM = 32
NUM_CORES = 2
NUM_SUBCORES = 16
NUM_WORKERS = NUM_CORES * NUM_SUBCORES
ROWS_PER_WORKER = BATCH // NUM_WORKERS
CHUNK = 128
NUM_CHUNKS = ROWS_PER_WORKER // CHUNK


def _gather_sc(uidx, iidx, emb_user, emb_item):
    mesh = plsc.VectorSubcoreMesh(core_axis_name="c", subcore_axis_name="s")

    @functools.partial(
        pl.kernel,
        out_type=[
            jax.ShapeDtypeStruct((BATCH, EMBED_DIM), jnp.float32),
            jax.ShapeDtypeStruct((BATCH, EMBED_DIM), jnp.float32),
        ],
        mesh=mesh,
        compiler_params=pltpu.CompilerParams(use_tc_tiling_on_sc=False),
        scratch_types=[
            pltpu.VMEM((NUM_CHUNKS, CHUNK), jnp.int32),
            pltpu.VMEM((NUM_CHUNKS, CHUNK), jnp.int32),
            pltpu.VMEM((ROWS_PER_WORKER, EMBED_DIM), jnp.float32),
            pltpu.VMEM((ROWS_PER_WORKER, EMBED_DIM), jnp.float32),
            pltpu.SemaphoreType.DMA,
        ],
    )
    def gather_kernel(uidx_hbm, iidx_hbm, ut_hbm, it_hbm, ue_hbm, ie_hbm,
                      uidx_v, iidx_v, ur_v, ir_v, sem):
        wid = lax.axis_index("s") * NUM_CORES + lax.axis_index("c")
        base = wid * ROWS_PER_WORKER
        pltpu.sync_copy(uidx_hbm.at[wid], uidx_v)
        pltpu.sync_copy(iidx_hbm.at[wid], iidx_v)
        copies = []
        for j in range(NUM_CHUNKS):
            copies.append(pltpu.async_copy(
                ut_hbm.at[uidx_v.at[j]],
                ur_v.at[pl.ds(j * CHUNK, CHUNK)], sem))
            copies.append(pltpu.async_copy(
                it_hbm.at[iidx_v.at[j]],
                ir_v.at[pl.ds(j * CHUNK, CHUNK)], sem))
        for c in copies:
            c.wait()
        pltpu.sync_copy(ur_v, ue_hbm.at[pl.ds(base, ROWS_PER_WORKER)])
        pltpu.sync_copy(ir_v, ie_hbm.at[pl.ds(base, ROWS_PER_WORKER)])

    return gather_kernel(uidx, iidx, emb_user, emb_item)


def _mlp_body(ue_ref, ie_ref, w1_ref, b1_ref, w2_ref, b2_ref, w3_ref, b3_ref,
              w4_ref, b4_ref, o_ref):
    h = (jnp.dot(ue_ref[...], w1_ref[0:EMBED_DIM, :],
                 preferred_element_type=jnp.float32)
         + jnp.dot(ie_ref[...], w1_ref[EMBED_DIM:2 * EMBED_DIM, :],
                   preferred_element_type=jnp.float32)
         + b1_ref[...])
    h = jnp.maximum(h, 0.0)
    h = jnp.maximum(jnp.dot(h, w2_ref[...], preferred_element_type=jnp.float32)
                    + b2_ref[...], 0.0)
    h = jnp.maximum(jnp.dot(h, w3_ref[...], preferred_element_type=jnp.float32)
                    + b3_ref[...], 0.0)
    y = jax.nn.sigmoid(jnp.dot(h, w4_ref[...], preferred_element_type=jnp.float32)
                       + b4_ref[...])
    o_ref[...] = y * 5.0 + 1.0


def _mlp_tc(ue, ie, W1, b1, W2, b2, W3, b3, W4, b4):
    blk = 2048
    grid = (BATCH // blk,)
    full = lambda shape: pl.BlockSpec(shape, lambda i: (0, 0))
    return pl.pallas_call(
        _mlp_body,
        grid=grid,
        in_specs=[
            pl.BlockSpec((blk, EMBED_DIM), lambda i: (i, 0)),
            pl.BlockSpec((blk, EMBED_DIM), lambda i: (i, 0)),
            full(W1.shape), full(b1.shape),
            full(W2.shape), full(b2.shape),
            full(W3.shape), full(b3.shape),
            full(W4.shape), full(b4.shape),
        ],
        out_specs=pl.BlockSpec((blk, 1), lambda i: (i, 0)),
        out_shape=jax.ShapeDtypeStruct((BATCH, 1), jnp.float32),
    )(ue, ie, W1, b1, W2, b2, W3, b3, W4, b4)


def kernel(user_indices, item_indices, emb_user, emb_item,
           W1, b1, W2, b2, W3, b3, W4, b4):
    uidx = user_indices.astype(jnp.int32).reshape(NUM_WORKERS, NUM_CHUNKS, CHUNK)
    iidx = item_indices.astype(jnp.int32).reshape(NUM_WORKERS, NUM_CHUNKS, CHUNK)
    ue, ie = _gather_sc(uidx, iidx, emb_user, emb_item)
    return _mlp_tc(ue, ie, W1, b1.reshape(1, -1), W2, b2.reshape(1, -1),
                   W3, b3.reshape(1, -1), W4, b4.reshape(1, -1))


# speedup vs baseline: 3.8350x; 2.3226x over previous
import functools

import jax
import jax.numpy as jnp
from jax import lax
from jax.experimental import pallas as pl
from jax.experimental.pallas import tpu as pltpu
from jax.experimental.pallas import tpu_sc as plsc

BATCH = 16384
EMBED_DIM = 32
WIDE = 128
NUM_USERS = 1000000
NUM_FILMS = 100000
NUM_CORES = 2
NUM_SUBCORES = 16
NUM_WORKERS = NUM_CORES * NUM_SUBCORES
LANES = 16
BLK = 512
RES = 128

U_FULL = NUM_USERS // BLK
I_FULL = NUM_FILMS // BLK
CUT_U = U_FULL * BLK
CUT_I = I_FULL * BLK
LCAP = BATCH + LANES


def _gather_sc(uidx2, iidx2, ut_t, it_t):
    mesh = plsc.VectorSubcoreMesh(core_axis_name="c", subcore_axis_name="s")

    @functools.partial(
        pl.kernel,
        out_type=[
            jax.ShapeDtypeStruct((BATCH + RES, WIDE), jnp.float32),
            jax.ShapeDtypeStruct((BATCH + RES, WIDE), jnp.float32),
        ],
        mesh=mesh,
        compiler_params=pltpu.CompilerParams(needs_layout_passes=False),
        scratch_types=[
            pltpu.VMEM((BATCH // WIDE, WIDE), jnp.int32),
            pltpu.VMEM((LCAP,), jnp.int32),
            pltpu.VMEM((LCAP,), jnp.int32),
            pltpu.VMEM((2, EMBED_DIM, BLK), jnp.float32),
            pltpu.VMEM((RES, WIDE), jnp.float32),
            pltpu.VMEM((RES,), jnp.int32),
            pltpu.SemaphoreType.DMA,
            pltpu.SemaphoreType.DMA,
            pltpu.SemaphoreType.DMA,
        ],
    )
    def gather_kernel(uidx_hbm, iidx_hbm, ut_hbm, it_hbm, ou_hbm, oi_hbm,
                      idx_v, lidx_v, lpos_v, blk_v, res_v, row_v,
                      isem, bsem, osem):
        wid = lax.axis_index("s") * NUM_CORES + lax.axis_index("c")
        iota = lax.iota(jnp.int32, LANES)

        def init_rows():
            for k in range(RES // LANES):
                row_v[pl.ds(k * LANES, LANES)] = BATCH + k * LANES + iota

        def run_phase(tab_hbm, sidx_hbm, out_hbm, nfull):
            per_lo = nfull // NUM_WORKERS
            nlo = nfull % NUM_WORKERS
            nblk = per_lo + jnp.where(wid < nlo, 1, 0)
            blk0 = wid * per_lo + jnp.minimum(wid, nlo)
            c0 = blk0 * BLK
            c1 = c0 + nblk * BLK

            pltpu.sync_copy(sidx_hbm, idx_v)
            init_rows()

            def prefilter(r, cnt):
                for k in range(WIDE // LANES):
                    g16 = idx_v[r, pl.ds(k * LANES, LANES)]
                    p16 = r * WIDE + k * LANES + iota
                    m = (g16 >= c0) & (g16 < c1)
                    plsc.store_compressed(lidx_v.at[pl.ds(cnt, LANES)], g16, mask=m)
                    plsc.store_compressed(lpos_v.at[pl.ds(cnt, LANES)], p16, mask=m)
                    cnt = cnt + jnp.sum(m.astype(jnp.int32))
                return cnt

            cnt = lax.fori_loop(0, BATCH // WIDE, prefilter,
                                jnp.int32(0), unroll=False)
            ngrp = (cnt + LANES - 1) // LANES

            def in_copy(b, slot):
                col0 = pl.multiple_of((blk0 + b) * BLK, BLK)
                return pltpu.make_async_copy(
                    tab_hbm.at[:, pl.ds(col0, BLK)], blk_v.at[slot], isem)

            def flush(fill):
                pltpu.async_copy(res_v, out_hbm.at[row_v], osem).wait()
                init_rows()
                return jnp.int32(0)

            def extract(slot, b0, g, fill):
                l16 = lidx_v[pl.ds(g * LANES, LANES)]
                p16 = lpos_v[pl.ds(g * LANES, LANES)]
                m = ((l16 >= b0) & (l16 < b0 + BLK)
                     & (g * LANES + iota < cnt))
                pop = jnp.sum(m.astype(jnp.int32))

                def hit(fill):
                    fill = lax.cond(fill + pop > RES, flush,
                                    lambda f: f, fill)
                    slots = fill + plsc.cumsum(m.astype(jnp.int32)) - 1
                    cols = jnp.clip(l16 - b0, 0, BLK - 1)
                    for c in range(EMBED_DIM):
                        vals = plsc.load_gather(
                            blk_v.at[slot],
                            [jnp.full((LANES,), c, jnp.int32), cols])
                        plsc.store_scatter(
                            res_v, [slots, jnp.full((LANES,), c, jnp.int32)],
                            vals, mask=m)
                    plsc.store_scatter(row_v, [slots], p16, mask=m)
                    return fill + pop

                return lax.cond(pop > 0, hit, lambda f: f, fill)

            def block_body(b, fill):
                slot = lax.rem(b, 2)
                in_copy(b, slot).wait()
                in_copy(jnp.minimum(b + 1, nblk - 1),
                        lax.rem(b + 1, 2)).start()
                b0 = (blk0 + b) * BLK

                def grp_body(g, fill):
                    return extract(slot, b0, g, fill)

                return lax.fori_loop(0, ngrp, grp_body, fill, unroll=False)

            in_copy(0, 0).start()
            fill = lax.fori_loop(0, nblk, block_body,
                                 jnp.int32(0), unroll=False)
            in_copy(nblk - 1, lax.rem(nblk, 2)).wait()
            flush(fill)

        run_phase(ut_hbm, uidx_hbm, ou_hbm, U_FULL)
        run_phase(it_hbm, iidx_hbm, oi_hbm, I_FULL)

    return gather_kernel(uidx2, iidx2, ut_t, it_t)


def _mlp_body(uw_ref, iw_ref, ur_ref, ir_ref, utail_ref, itail_ref,
              w1_ref, b1_ref, w2_ref, b2_ref,
              w3_ref, b3_ref, w4_ref, b4_ref, o_ref):
    ue = uw_ref[:, 0:EMBED_DIM]
    ie = iw_ref[:, 0:EMBED_DIM]
    ur = ur_ref[...]
    ir = ir_ref[...]
    oh_u = (ur - CUT_U == lax.broadcasted_iota(jnp.int32, (1, NUM_USERS - CUT_U), 1)
            ).astype(jnp.float32)
    oh_i = (ir - CUT_I == lax.broadcasted_iota(jnp.int32, (1, NUM_FILMS - CUT_I), 1)
            ).astype(jnp.float32)
    ue_tail = jnp.dot(oh_u, utail_ref[...], preferred_element_type=jnp.float32)
    ie_tail = jnp.dot(oh_i, itail_ref[...], preferred_element_type=jnp.float32)
    ue = jnp.where(ur < CUT_U, ue, ue_tail)
    ie = jnp.where(ir < CUT_I, ie, ie_tail)
    h = (jnp.dot(ue, w1_ref[0:EMBED_DIM, :], preferred_element_type=jnp.float32)
         + jnp.dot(ie, w1_ref[EMBED_DIM:2 * EMBED_DIM, :],
                   preferred_element_type=jnp.float32)
         + b1_ref[...])
    h = jnp.maximum(h, 0.0)
    h = jnp.maximum(jnp.dot(h, w2_ref[...], preferred_element_type=jnp.float32)
                    + b2_ref[...], 0.0)
    h = jnp.maximum(jnp.dot(h, w3_ref[...], preferred_element_type=jnp.float32)
                    + b3_ref[...], 0.0)
    y = jax.nn.sigmoid(jnp.dot(h, w4_ref[...], preferred_element_type=jnp.float32)
                       + b4_ref[...])
    o_ref[...] = y * 5.0 + 1.0


def _mlp_tc(uw, iw, ur, ir, utail, itail, W1, b1, W2, b2, W3, b3, W4, b4):
    blk = 2048
    grid = (BATCH // blk,)
    full = lambda shape: pl.BlockSpec(shape, lambda i: (0,) * len(shape))
    return pl.pallas_call(
        _mlp_body,
        grid=grid,
        in_specs=[
            pl.BlockSpec((blk, WIDE), lambda i: (i, 0)),
            pl.BlockSpec((blk, WIDE), lambda i: (i, 0)),
            pl.BlockSpec((blk, 1), lambda i: (i, 0)),
            pl.BlockSpec((blk, 1), lambda i: (i, 0)),
            full(utail.shape), full(itail.shape),
            full(W1.shape), full(b1.shape),
            full(W2.shape), full(b2.shape),
            full(W3.shape), full(b3.shape),
            full(W4.shape), full(b4.shape),
        ],
        out_specs=pl.BlockSpec((blk, 1), lambda i: (i, 0)),
        out_shape=jax.ShapeDtypeStruct((BATCH, 1), jnp.float32),
    )(uw, iw, ur, ir, utail, itail, W1, b1, W2, b2, W3, b3, W4, b4)


def kernel(user_indices, item_indices, emb_user, emb_item,
           W1, b1, W2, b2, W3, b3, W4, b4):
    ui = user_indices.astype(jnp.int32)
    ii = item_indices.astype(jnp.int32)
    uw, iw = _gather_sc(ui.reshape(BATCH // WIDE, WIDE),
                        ii.reshape(BATCH // WIDE, WIDE),
                        emb_user.T, emb_item.T)
    return _mlp_tc(uw, iw, ui.reshape(BATCH, 1), ii.reshape(BATCH, 1),
                   emb_user[CUT_U:], emb_item[CUT_I:],
                   W1, b1.reshape(1, -1), W2, b2.reshape(1, -1),
                   W3, b3.reshape(1, -1), W4, b4.reshape(1, -1))


# speedup vs baseline: 3.8705x; 1.0093x over previous
import functools

import jax
import jax.numpy as jnp
from jax import lax
from jax.experimental import pallas as pl
from jax.experimental.pallas import tpu as pltpu
from jax.experimental.pallas import tpu_sc as plsc

BATCH = 16384
EMBED_DIM = 32
WIDE = 128
NUM_USERS = 1000000
NUM_FILMS = 100000
NUM_CORES = 2
NUM_SUBCORES = 16
NUM_WORKERS = NUM_CORES * NUM_SUBCORES
LANES = 16
BLK = 512
RES = 128

U_FULL = NUM_USERS // BLK
I_FULL = NUM_FILMS // BLK
CUT_U = U_FULL * BLK
CUT_I = I_FULL * BLK
LCAP = BATCH + LANES


def _gather_sc(uidx2, iidx2, ut_t, it_t):
    mesh = plsc.VectorSubcoreMesh(core_axis_name="c", subcore_axis_name="s")

    @functools.partial(
        pl.kernel,
        out_type=[
            jax.ShapeDtypeStruct((BATCH + RES, WIDE), jnp.float32),
            jax.ShapeDtypeStruct((BATCH + RES, WIDE), jnp.float32),
        ],
        mesh=mesh,
        compiler_params=pltpu.CompilerParams(needs_layout_passes=False),
        scratch_types=[
            pltpu.VMEM((BATCH // WIDE, WIDE), jnp.int32),
            pltpu.VMEM((LCAP,), jnp.int32),
            pltpu.VMEM((LCAP,), jnp.int32),
            pltpu.VMEM((2, EMBED_DIM, BLK), jnp.float32),
            pltpu.VMEM((RES, WIDE), jnp.float32),
            pltpu.VMEM((RES,), jnp.int32),
            pltpu.SemaphoreType.DMA,
            pltpu.SemaphoreType.DMA,
            pltpu.SemaphoreType.DMA,
        ],
    )
    def gather_kernel(uidx_hbm, iidx_hbm, ut_hbm, it_hbm, ou_hbm, oi_hbm,
                      idx_v, lidx_v, lpos_v, blk_v, res_v, row_v,
                      isem, bsem, osem):
        wid = lax.axis_index("s") * NUM_CORES + lax.axis_index("c")
        iota = lax.iota(jnp.int32, LANES)

        def init_rows():
            for k in range(RES // LANES):
                row_v[pl.ds(k * LANES, LANES)] = BATCH + k * LANES + iota

        def run_phase(tab_hbm, sidx_hbm, out_hbm, nfull):
            per_lo = nfull // NUM_WORKERS
            nlo = nfull % NUM_WORKERS
            nblk = per_lo + jnp.where(wid < nlo, 1, 0)
            blk0 = wid * per_lo + jnp.minimum(wid, nlo)
            c0 = blk0 * BLK
            c1 = c0 + nblk * BLK

            pltpu.sync_copy(sidx_hbm, idx_v)
            init_rows()

            def prefilter(r, cnt):
                for k in range(WIDE // LANES):
                    g16 = idx_v[r, pl.ds(k * LANES, LANES)]
                    p16 = r * WIDE + k * LANES + iota
                    m = (g16 >= c0) & (g16 < c1)
                    plsc.store_compressed(lidx_v.at[pl.ds(cnt, LANES)], g16, mask=m)
                    plsc.store_compressed(lpos_v.at[pl.ds(cnt, LANES)], p16, mask=m)
                    cnt = cnt + jnp.sum(m.astype(jnp.int32))
                return cnt

            cnt = lax.fori_loop(0, BATCH // WIDE, prefilter,
                                jnp.int32(0), unroll=False)
            lidx_v[pl.ds(cnt, LANES)] = jnp.full((LANES,), 2**30, jnp.int32)
            ngrp = (cnt + LANES - 1) // LANES

            def in_copy(b, slot):
                col0 = pl.multiple_of((blk0 + b) * BLK, BLK)
                return pltpu.make_async_copy(
                    tab_hbm.at[:, pl.ds(col0, BLK)], blk_v.at[slot], isem)

            def flush(fill):
                pltpu.async_copy(res_v, out_hbm.at[row_v], osem).wait()
                init_rows()
                return jnp.int32(0)

            def extract(slot, b0, g, fill):
                l16 = lidx_v[pl.ds(g * LANES, LANES)]
                p16 = lpos_v[pl.ds(g * LANES, LANES)]
                m = (l16 >= b0) & (l16 < b0 + BLK)
                pop = jnp.sum(m.astype(jnp.int32))

                def hit(fill):
                    fill = lax.cond(fill + pop > RES, flush,
                                    lambda f: f, fill)
                    slots = fill + plsc.cumsum(m.astype(jnp.int32)) - 1
                    cols = jnp.clip(l16 - b0, 0, BLK - 1)
                    for c in range(EMBED_DIM):
                        vals = plsc.load_gather(
                            blk_v.at[slot],
                            [jnp.full((LANES,), c, jnp.int32), cols])
                        plsc.store_scatter(
                            res_v, [slots, jnp.full((LANES,), c, jnp.int32)],
                            vals, mask=m)
                    plsc.store_scatter(row_v, [slots], p16, mask=m)
                    return fill + pop

                return lax.cond(pop > 0, hit, lambda f: f, fill)

            def block_body(b, fill):
                slot = lax.rem(b, 2)
                in_copy(b, slot).wait()
                in_copy(jnp.minimum(b + 1, nblk - 1),
                        lax.rem(b + 1, 2)).start()
                b0 = (blk0 + b) * BLK

                def grp_body(g, fill):
                    return extract(slot, b0, g, fill)

                return lax.fori_loop(0, ngrp, grp_body, fill, unroll=False)

            in_copy(0, 0).start()
            fill = lax.fori_loop(0, nblk, block_body,
                                 jnp.int32(0), unroll=False)
            in_copy(nblk - 1, lax.rem(nblk, 2)).wait()
            flush(fill)

        run_phase(ut_hbm, uidx_hbm, ou_hbm, U_FULL)
        run_phase(it_hbm, iidx_hbm, oi_hbm, I_FULL)

    return gather_kernel(uidx2, iidx2, ut_t, it_t)


def _mlp_body(uw_ref, iw_ref, ur_ref, ir_ref, utail_ref, itail_ref,
              w1_ref, b1_ref, w2_ref, b2_ref,
              w3_ref, b3_ref, w4_ref, b4_ref, o_ref):
    ue = uw_ref[:, 0:EMBED_DIM]
    ie = iw_ref[:, 0:EMBED_DIM]
    ur = ur_ref[...]
    ir = ir_ref[...]
    oh_u = (ur - CUT_U == lax.broadcasted_iota(jnp.int32, (1, NUM_USERS - CUT_U), 1)
            ).astype(jnp.float32)
    oh_i = (ir - CUT_I == lax.broadcasted_iota(jnp.int32, (1, NUM_FILMS - CUT_I), 1)
            ).astype(jnp.float32)
    ue_tail = jnp.dot(oh_u, utail_ref[...], preferred_element_type=jnp.float32)
    ie_tail = jnp.dot(oh_i, itail_ref[...], preferred_element_type=jnp.float32)
    ue = jnp.where(ur < CUT_U, ue, ue_tail)
    ie = jnp.where(ir < CUT_I, ie, ie_tail)
    h = (jnp.dot(ue, w1_ref[0:EMBED_DIM, :], preferred_element_type=jnp.float32)
         + jnp.dot(ie, w1_ref[EMBED_DIM:2 * EMBED_DIM, :],
                   preferred_element_type=jnp.float32)
         + b1_ref[...])
    h = jnp.maximum(h, 0.0)
    h = jnp.maximum(jnp.dot(h, w2_ref[...], preferred_element_type=jnp.float32)
                    + b2_ref[...], 0.0)
    h = jnp.maximum(jnp.dot(h, w3_ref[...], preferred_element_type=jnp.float32)
                    + b3_ref[...], 0.0)
    y = jax.nn.sigmoid(jnp.dot(h, w4_ref[...], preferred_element_type=jnp.float32)
                       + b4_ref[...])
    o_ref[...] = y * 5.0 + 1.0


def _mlp_tc(uw, iw, ur, ir, utail, itail, W1, b1, W2, b2, W3, b3, W4, b4):
    blk = 2048
    grid = (BATCH // blk,)
    full = lambda shape: pl.BlockSpec(shape, lambda i: (0,) * len(shape))
    return pl.pallas_call(
        _mlp_body,
        grid=grid,
        in_specs=[
            pl.BlockSpec((blk, WIDE), lambda i: (i, 0)),
            pl.BlockSpec((blk, WIDE), lambda i: (i, 0)),
            pl.BlockSpec((blk, 1), lambda i: (i, 0)),
            pl.BlockSpec((blk, 1), lambda i: (i, 0)),
            full(utail.shape), full(itail.shape),
            full(W1.shape), full(b1.shape),
            full(W2.shape), full(b2.shape),
            full(W3.shape), full(b3.shape),
            full(W4.shape), full(b4.shape),
        ],
        out_specs=pl.BlockSpec((blk, 1), lambda i: (i, 0)),
        out_shape=jax.ShapeDtypeStruct((BATCH, 1), jnp.float32),
    )(uw, iw, ur, ir, utail, itail, W1, b1, W2, b2, W3, b3, W4, b4)


def kernel(user_indices, item_indices, emb_user, emb_item,
           W1, b1, W2, b2, W3, b3, W4, b4):
    ui = user_indices.astype(jnp.int32)
    ii = item_indices.astype(jnp.int32)
    uw, iw = _gather_sc(ui.reshape(BATCH // WIDE, WIDE),
                        ii.reshape(BATCH // WIDE, WIDE),
                        emb_user.T, emb_item.T)
    return _mlp_tc(uw, iw, ui.reshape(BATCH, 1), ii.reshape(BATCH, 1),
                   emb_user[CUT_U:], emb_item[CUT_I:],
                   W1, b1.reshape(1, -1), W2, b2.reshape(1, -1),
                   W3, b3.reshape(1, -1), W4, b4.reshape(1, -1))
